# Initial kernel scaffold; baseline (speedup 1.0000x reference)
import jax
import jax.numpy as jnp
from jax.experimental import pallas as pl


def kernel(X_w, X_d, emb_tables, W1, b1, W2, b2, W3, b3, Wfc, bfc):
    raise NotImplementedError("write your pallas kernel here")


# speedup vs baseline: 10.2588x; 10.2588x over previous
import functools

import jax
import jax.numpy as jnp
from jax import lax
from jax.experimental import pallas as pl
from jax.experimental.pallas import tpu as pltpu
from jax.experimental.pallas import tpu_sc as plsc

B = 4096
F = 26
V = 1000
D = 128
H1, H2, H3 = 1024, 512, 256
WIDE = 1024
DEEP_DIM = F * D

NC, NS = 2, 16
NW = NC * NS
ROWS = B * F
RPW = ROWS // NW
CHUNK = 128
NCH = RPW // CHUNK

BT = 512


def _gather_sc(table, idx):
    mesh = plsc.VectorSubcoreMesh(
        core_axis_name="c", subcore_axis_name="s",
        num_cores=NC, num_subcores=NS)

    @functools.partial(
        pl.kernel,
        out_type=jax.ShapeDtypeStruct((NW * NCH, CHUNK, D), jnp.float32),
        mesh=mesh,
        scratch_types=[
            pltpu.VMEM((NCH, CHUNK), jnp.int32),
            pltpu.VMEM((CHUNK, D), jnp.float32),
            pltpu.SemaphoreType.DMA,
        ],
    )
    def k(table_hbm, idx_hbm, out_hbm, idx_v, rows_v, sem):
        wid = lax.axis_index("s") * NC + lax.axis_index("c")
        pltpu.sync_copy(idx_hbm.at[wid], idx_v)

        def body(c, carry):
            pltpu.async_copy(table_hbm.at[idx_v.at[c]], rows_v, sem).wait()
            pltpu.sync_copy(rows_v, out_hbm.at[wid * NCH + c])
            return carry

        lax.fori_loop(0, NCH, body, 0)

    return k(table, idx)


def _mlp_body(xw_ref, emb_ref, w1_ref, b1_ref, w2_ref, b2_ref, w3_ref,
              b3_ref, wfcw_ref, wfcd_ref, bfc_ref, out_ref):
    h = jnp.dot(emb_ref[...].astype(jnp.bfloat16), w1_ref[...],
                preferred_element_type=jnp.float32)
    h = jnp.maximum(h + b1_ref[...], 0.0).astype(jnp.bfloat16)
    h = jnp.dot(h, w2_ref[...], preferred_element_type=jnp.float32)
    h = jnp.maximum(h + b2_ref[...], 0.0).astype(jnp.bfloat16)
    h = jnp.dot(h, w3_ref[...], preferred_element_type=jnp.float32)
    h = jnp.maximum(h + b3_ref[...], 0.0)
    out = jnp.dot(xw_ref[...], wfcw_ref[...], preferred_element_type=jnp.float32)
    out = out + jnp.dot(h, wfcd_ref[...], preferred_element_type=jnp.float32)
    out_ref[...] = out + bfc_ref[...]


def _mlp_tc(xw, emb, w1, b1, w2, b2, w3, b3, wfcw, wfcd, bfc):
    grid = (B // BT,)
    return pl.pallas_call(
        _mlp_body,
        grid=grid,
        in_specs=[
            pl.BlockSpec((BT, WIDE), lambda i: (i, 0)),
            pl.BlockSpec((BT, DEEP_DIM), lambda i: (i, 0)),
            pl.BlockSpec((DEEP_DIM, H1), lambda i: (0, 0)),
            pl.BlockSpec((1, H1), lambda i: (0, 0)),
            pl.BlockSpec((H1, H2), lambda i: (0, 0)),
            pl.BlockSpec((1, H2), lambda i: (0, 0)),
            pl.BlockSpec((H2, H3), lambda i: (0, 0)),
            pl.BlockSpec((1, H3), lambda i: (0, 0)),
            pl.BlockSpec((WIDE, 1), lambda i: (0, 0)),
            pl.BlockSpec((H3, 1), lambda i: (0, 0)),
            pl.BlockSpec((1, 1), lambda i: (0, 0)),
        ],
        out_specs=pl.BlockSpec((BT, 1), lambda i: (i, 0)),
        out_shape=jax.ShapeDtypeStruct((B, 1), jnp.float32),
    )(xw, emb, w1, b1, w2, b2, w3, b3, wfcw, wfcd, bfc)


def kernel(X_w, X_d, emb_tables, W1, b1, W2, b2, W3, b3, Wfc, bfc):
    table = emb_tables.reshape(F * V, D)
    idx = (X_d.astype(jnp.int32) + (jnp.arange(F, dtype=jnp.int32) * V)[None, :])
    idx = idx.reshape(NW, NCH, CHUNK)
    emb = _gather_sc(table, idx).reshape(B, DEEP_DIM)
    out = _mlp_tc(
        X_w, emb,
        W1.astype(jnp.bfloat16), b1.reshape(1, H1),
        W2.astype(jnp.bfloat16), b2.reshape(1, H2),
        W3.astype(jnp.bfloat16), b3.reshape(1, H3),
        Wfc[:WIDE], Wfc[WIDE:], bfc.reshape(1, 1),
    )
    return out


# speedup vs baseline: 15.7320x; 1.5335x over previous
import functools

import jax
import jax.numpy as jnp
from jax import lax
from jax.experimental import pallas as pl
from jax.experimental.pallas import tpu as pltpu
from jax.experimental.pallas import tpu_sc as plsc

B = 4096
F = 26
V = 1000
D = 128
H1, H2, H3 = 1024, 512, 256
WIDE = 1024
DEEP_DIM = F * D

NC, NS = 2, 16
NW = NC * NS
ROWS = B * F
RPW = ROWS // NW
CHUNK = 128
NCH = RPW // CHUNK

BT = 512


def _gather_sc(table, idx):
    mesh = plsc.VectorSubcoreMesh(
        core_axis_name="c", subcore_axis_name="s",
        num_cores=NC, num_subcores=NS)

    @functools.partial(
        pl.kernel,
        out_type=jax.ShapeDtypeStruct((B, DEEP_DIM), jnp.float32),
        mesh=mesh,
        scratch_types=[
            pltpu.VMEM((F, CHUNK), jnp.int32),
            pltpu.VMEM((CHUNK, D), jnp.float32),
            pltpu.VMEM((CHUNK, D), jnp.float32),
            pltpu.SemaphoreType.DMA,
            pltpu.SemaphoreType.DMA,
            pltpu.SemaphoreType.DMA,
            pltpu.SemaphoreType.DMA,
        ],
    )
    def k(table_hbm, idx_hbm, out_hbm, idx_v, buf_a, buf_b, sga, sgb, ssa, ssb):
        wid = lax.axis_index("s") * NC + lax.axis_index("c")
        row0 = pl.multiple_of(wid * CHUNK, CHUNK)
        pltpu.sync_copy(idx_hbm.at[wid], idx_v)

        def g_start(c, buf, sem):
            pltpu.make_async_copy(table_hbm.at[idx_v.at[c]], buf, sem).start()

        def g_wait(c, buf, sem):
            pltpu.make_async_copy(table_hbm.at[idx_v.at[c]], buf, sem).wait()

        def out_block(c):
            col = pl.multiple_of(c * D, D)
            return out_hbm.at[pl.ds(row0, CHUNK), pl.ds(col, D)]

        def s_start(c, buf, sem):
            pltpu.make_async_copy(buf, out_block(c), sem).start()

        def s_wait(c, buf, sem):
            pltpu.make_async_copy(buf, out_block(c), sem).wait()

        g_start(0, buf_a, sga)
        g_start(1, buf_b, sgb)

        def body(i, carry):
            c0 = 2 * i
            c1 = 2 * i + 1
            g_wait(c0, buf_a, sga)
            s_start(c0, buf_a, ssa)
            g_wait(c1, buf_b, sgb)
            s_start(c1, buf_b, ssb)
            s_wait(c0, buf_a, ssa)

            @pl.when(i < NCH // 2 - 1)
            def _():
                g_start(c0 + 2, buf_a, sga)

            s_wait(c1, buf_b, ssb)

            @pl.when(i < NCH // 2 - 1)
            def _():
                g_start(c1 + 2, buf_b, sgb)

            return carry

        lax.fori_loop(0, NCH // 2, body, 0)

    return k(table, idx)


def _mlp_body(xw_ref, emb_ref, w1_ref, b1_ref, w2_ref, b2_ref, w3_ref,
              b3_ref, wfcw_ref, wfcd_ref, bfc_ref, out_ref):
    h = jnp.dot(emb_ref[...].astype(jnp.bfloat16), w1_ref[...],
                preferred_element_type=jnp.float32)
    h = jnp.maximum(h + b1_ref[...], 0.0).astype(jnp.bfloat16)
    h = jnp.dot(h, w2_ref[...], preferred_element_type=jnp.float32)
    h = jnp.maximum(h + b2_ref[...], 0.0).astype(jnp.bfloat16)
    h = jnp.dot(h, w3_ref[...], preferred_element_type=jnp.float32)
    h = jnp.maximum(h + b3_ref[...], 0.0)
    out = jnp.dot(xw_ref[...], wfcw_ref[...], preferred_element_type=jnp.float32)
    out = out + jnp.dot(h, wfcd_ref[...], preferred_element_type=jnp.float32)
    out_ref[...] = out + bfc_ref[...]


def _mlp_tc(xw, emb, w1, b1, w2, b2, w3, b3, wfcw, wfcd, bfc):
    grid = (B // BT,)
    return pl.pallas_call(
        _mlp_body,
        grid=grid,
        in_specs=[
            pl.BlockSpec((BT, WIDE), lambda i: (i, 0)),
            pl.BlockSpec((BT, DEEP_DIM), lambda i: (i, 0)),
            pl.BlockSpec((DEEP_DIM, H1), lambda i: (0, 0)),
            pl.BlockSpec((1, H1), lambda i: (0, 0)),
            pl.BlockSpec((H1, H2), lambda i: (0, 0)),
            pl.BlockSpec((1, H2), lambda i: (0, 0)),
            pl.BlockSpec((H2, H3), lambda i: (0, 0)),
            pl.BlockSpec((1, H3), lambda i: (0, 0)),
            pl.BlockSpec((WIDE, 1), lambda i: (0, 0)),
            pl.BlockSpec((H3, 1), lambda i: (0, 0)),
            pl.BlockSpec((1, 1), lambda i: (0, 0)),
        ],
        out_specs=pl.BlockSpec((BT, 1), lambda i: (i, 0)),
        out_shape=jax.ShapeDtypeStruct((B, 1), jnp.float32),
    )(xw, emb, w1, b1, w2, b2, w3, b3, wfcw, wfcd, bfc)


def kernel(X_w, X_d, emb_tables, W1, b1, W2, b2, W3, b3, Wfc, bfc):
    table = emb_tables.reshape(F * V, D)
    idx = X_d.astype(jnp.int32).reshape(NW, CHUNK, F).transpose(0, 2, 1)
    idx = idx + (jnp.arange(F, dtype=jnp.int32) * V)[None, :, None]
    emb = _gather_sc(table, idx)
    out = _mlp_tc(
        X_w, emb,
        W1.astype(jnp.bfloat16), b1.reshape(1, H1),
        W2.astype(jnp.bfloat16), b2.reshape(1, H2),
        W3.astype(jnp.bfloat16), b3.reshape(1, H3),
        Wfc[:WIDE], Wfc[WIDE:], bfc.reshape(1, 1),
    )
    return out
